# baseline (speedup 1.0000x reference)
import functools
from functools import partial

import jax
import jax.numpy as jnp
import numpy as np
from jax.experimental import pallas as pl
from jax.experimental.pallas import tpu as pltpu

IRREPS = [(128, 0, 1), (64, 1, -1), (32, 2, 1)]
G = 256
EPS = 1e-05
DIM = 480
NMUL = 224
NSC = 128
SW = 640
BN = 400


def _stats_kernel(x_ref, b_ref, o_ref, acc):
    i = pl.program_id(0)

    @pl.when(i == 0)
    def _():
        acc[...] = jnp.zeros_like(acc)

    x = x_ref[...]
    seg = b_ref[0, 0, :]
    iota = jax.lax.broadcasted_iota(jnp.int32, (G, BN), 0)
    oh_t = (iota == seg[None, :]).astype(jnp.bfloat16)
    ones = jnp.ones((BN, 32), jnp.float32)
    y = jnp.concatenate([x[:, :NSC], x * x, ones], axis=1)
    y_hi = y.astype(jnp.bfloat16)
    y_lo = (y - y_hi.astype(jnp.float32)).astype(jnp.bfloat16)
    acc[...] += (
        jax.lax.dot_general(oh_t, y_hi, (((1,), (0,)), ((), ())),
                            preferred_element_type=jnp.float32)
        + jax.lax.dot_general(oh_t, y_lo, (((1,), (0,)), ((), ())),
                              preferred_element_type=jnp.float32))

    @pl.when(i == pl.num_programs(0) - 1)
    def _():
        o_ref[...] = acc[...]


def _table_kernel(st_ref, gm_ref, em_ref, dinv_ref, w_ref, ms_ref, bias_ref,
                  t_ref):
    st = st_ref[...]
    cnt = jnp.maximum(st[:, SW - 32:SW - 31], 1.0)
    s1 = st[:, :NSC]
    sq = st[:, NSC:NSC + DIM]
    m = s1 / cnt
    gq = jax.lax.dot_general(sq, gm_ref[...], (((1,), (0,)), ((), ())),
                             preferred_element_type=jnp.float32)
    ex2 = gq * dinv_ref[...] / cnt
    ms = ms_ref[...]
    corr = (m * m) * (ms * (2.0 - ms))
    corr_p = jnp.concatenate([corr, jnp.zeros((G, NMUL - NSC), jnp.float32)],
                             axis=1)
    fn = jax.lax.rsqrt(ex2 - corr_p + EPS) * w_ref[...]
    scale = jax.lax.dot_general(fn, em_ref[...], (((1,), (0,)), ((), ())),
                                preferred_element_type=jnp.float32)
    off_sc = bias_ref[...] - m * ms * fn[:, :NSC]
    off = jnp.concatenate([off_sc, jnp.zeros((G, DIM - NSC), jnp.float32)],
                          axis=1)
    t_ref[...] = jnp.concatenate([scale, off], axis=1)


def _apply_kernel(x_ref, b_ref, t_ref, o_ref):
    x = x_ref[...]
    seg = b_ref[0, 0, :]
    iota = jax.lax.broadcasted_iota(jnp.int32, (BN, G), 1)
    oh = (iota == seg[:, None]).astype(jnp.bfloat16)
    t = t_ref[...]
    t_hi = t.astype(jnp.bfloat16)
    t_lo = (t - t_hi.astype(jnp.float32)).astype(jnp.bfloat16)
    so = (jax.lax.dot_general(oh, t_hi, (((1,), (0,)), ((), ())),
                              preferred_element_type=jnp.float32)
          + jax.lax.dot_general(oh, t_lo, (((1,), (0,)), ((), ())),
                                preferred_element_type=jnp.float32))
    o_ref[...] = x * so[:, :DIM] + so[:, DIM:]


def _build_consts():
    d_of = np.concatenate([np.full(mul, 2 * l + 1, np.float32)
                           for mul, l, p in IRREPS])
    gm = np.zeros((DIM, NMUL), np.float32)
    em = np.zeros((NMUL, DIM), np.float32)
    c = 0
    mi = 0
    for mul, l, p in IRREPS:
        d = 2 * l + 1
        for k in range(mul):
            gm[c:c + d, mi] = 1.0
            em[mi, c:c + d] = 1.0
            c += d
            mi += 1
    dinv = (1.0 / d_of)[None, :]
    return jnp.asarray(gm), jnp.asarray(em), jnp.asarray(dinv)


_GM, _EM, _DINV = _build_consts()


@jax.jit
def kernel(node_input, batch, mean_shift, affine_weight, affine_bias):
    n = node_input.shape[0]
    nb = n // BN
    x = node_input
    b3 = batch.astype(jnp.int32).reshape(nb, 1, BN)

    stats = pl.pallas_call(
        _stats_kernel,
        grid=(nb,),
        in_specs=[
            pl.BlockSpec((BN, DIM), lambda i: (i, 0)),
            pl.BlockSpec((1, 1, BN), lambda i: (i, 0, 0)),
        ],
        out_specs=pl.BlockSpec((G, SW), lambda i: (0, 0)),
        out_shape=jax.ShapeDtypeStruct((G, SW), jnp.float32),
        scratch_shapes=[pltpu.VMEM((G, SW), jnp.float32)],
    )(x, b3)

    table = pl.pallas_call(
        _table_kernel,
        out_shape=jax.ShapeDtypeStruct((G, 2 * DIM), jnp.float32),
    )(stats, _GM, _EM, _DINV,
      affine_weight[None, :], mean_shift[None, :], affine_bias[None, :])

    out = pl.pallas_call(
        _apply_kernel,
        grid=(nb,),
        in_specs=[
            pl.BlockSpec((BN, DIM), lambda i: (i, 0)),
            pl.BlockSpec((1, 1, BN), lambda i: (i, 0, 0)),
            pl.BlockSpec((G, 2 * DIM), lambda i: (0, 0)),
        ],
        out_specs=pl.BlockSpec((BN, DIM), lambda i: (i, 0)),
        out_shape=jax.ShapeDtypeStruct((n, DIM), jnp.float32),
    )(x, b3, table)

    return out


# speedup vs baseline: 1.2665x; 1.2665x over previous
import functools
from functools import partial

import jax
import jax.numpy as jnp
import numpy as np
from jax.experimental import pallas as pl
from jax.experimental.pallas import tpu as pltpu

IRREPS = [(128, 0, 1), (64, 1, -1), (32, 2, 1)]
G = 256
EPS = 1e-05
DIM = 480
NMUL = 224
NSC = 128
SW = 640
BN = 1000


def _stats_kernel(x_ref, b_ref, o_ref, acc):
    i = pl.program_id(0)

    @pl.when(i == 0)
    def _():
        acc[...] = jnp.zeros_like(acc)

    x = x_ref[...]
    seg = b_ref[0, 0, :]
    iota = jax.lax.broadcasted_iota(jnp.int32, (G, BN), 0)
    oh_t = (iota == seg[None, :]).astype(jnp.bfloat16)
    ones = jnp.ones((BN, 32), jnp.float32)
    y = jnp.concatenate([x[:, :NSC], x * x, ones], axis=1)
    y_hi = y.astype(jnp.bfloat16)
    y_lo = (y - y_hi.astype(jnp.float32)).astype(jnp.bfloat16)
    acc[...] += (
        jax.lax.dot_general(oh_t, y_hi, (((1,), (0,)), ((), ())),
                            preferred_element_type=jnp.float32)
        + jax.lax.dot_general(oh_t, y_lo, (((1,), (0,)), ((), ())),
                              preferred_element_type=jnp.float32))

    @pl.when(i == pl.num_programs(0) - 1)
    def _():
        o_ref[...] = acc[...]


def _table_kernel(st_ref, gm_ref, em_ref, dinv_ref, w_ref, ms_ref, bias_ref,
                  t_ref):
    st = st_ref[...]
    cnt = jnp.maximum(st[:, SW - 32:SW - 31], 1.0)
    s1 = st[:, :NSC]
    sq = st[:, NSC:NSC + DIM]
    m = s1 / cnt
    gq = jax.lax.dot_general(sq, gm_ref[...], (((1,), (0,)), ((), ())),
                             preferred_element_type=jnp.float32)
    ex2 = gq * dinv_ref[...] / cnt
    ms = ms_ref[...]
    corr = (m * m) * (ms * (2.0 - ms))
    corr_p = jnp.concatenate([corr, jnp.zeros((G, NMUL - NSC), jnp.float32)],
                             axis=1)
    fn = jax.lax.rsqrt(ex2 - corr_p + EPS) * w_ref[...]
    scale = jax.lax.dot_general(fn, em_ref[...], (((1,), (0,)), ((), ())),
                                preferred_element_type=jnp.float32)
    off_sc = bias_ref[...] - m * ms * fn[:, :NSC]
    off = jnp.concatenate([off_sc, jnp.zeros((G, DIM - NSC), jnp.float32)],
                          axis=1)
    t_ref[...] = jnp.concatenate([scale, off], axis=1)


def _apply_kernel(x_ref, b_ref, t_ref, o_ref):
    x = x_ref[...]
    seg = b_ref[0, 0, :]
    iota = jax.lax.broadcasted_iota(jnp.int32, (BN, G), 1)
    oh = (iota == seg[:, None]).astype(jnp.bfloat16)
    t = t_ref[...]
    t_hi = t.astype(jnp.bfloat16)
    t_lo = (t - t_hi.astype(jnp.float32)).astype(jnp.bfloat16)
    so = (jax.lax.dot_general(oh, t_hi, (((1,), (0,)), ((), ())),
                              preferred_element_type=jnp.float32)
          + jax.lax.dot_general(oh, t_lo, (((1,), (0,)), ((), ())),
                                preferred_element_type=jnp.float32))
    o_ref[...] = x * so[:, :DIM] + so[:, DIM:]


def _build_consts():
    d_of = np.concatenate([np.full(mul, 2 * l + 1, np.float32)
                           for mul, l, p in IRREPS])
    gm = np.zeros((DIM, NMUL), np.float32)
    em = np.zeros((NMUL, DIM), np.float32)
    c = 0
    mi = 0
    for mul, l, p in IRREPS:
        d = 2 * l + 1
        for k in range(mul):
            gm[c:c + d, mi] = 1.0
            em[mi, c:c + d] = 1.0
            c += d
            mi += 1
    dinv = (1.0 / d_of)[None, :]
    return gm, em, dinv


_GM, _EM, _DINV = _build_consts()


@jax.jit
def kernel(node_input, batch, mean_shift, affine_weight, affine_bias):
    n = node_input.shape[0]
    nb = n // BN
    x = node_input
    b3 = batch.astype(jnp.int32).reshape(nb, 1, BN)

    stats = pl.pallas_call(
        _stats_kernel,
        grid=(nb,),
        in_specs=[
            pl.BlockSpec((BN, DIM), lambda i: (i, 0)),
            pl.BlockSpec((1, 1, BN), lambda i: (i, 0, 0)),
        ],
        out_specs=pl.BlockSpec((G, SW), lambda i: (0, 0)),
        out_shape=jax.ShapeDtypeStruct((G, SW), jnp.float32),
        scratch_shapes=[pltpu.VMEM((G, SW), jnp.float32)],
    )(x, b3)

    table = pl.pallas_call(
        _table_kernel,
        out_shape=jax.ShapeDtypeStruct((G, 2 * DIM), jnp.float32),
    )(stats, _GM, _EM, _DINV,
      affine_weight[None, :], mean_shift[None, :], affine_bias[None, :])

    out = pl.pallas_call(
        _apply_kernel,
        grid=(nb,),
        in_specs=[
            pl.BlockSpec((BN, DIM), lambda i: (i, 0)),
            pl.BlockSpec((1, 1, BN), lambda i: (i, 0, 0)),
            pl.BlockSpec((G, 2 * DIM), lambda i: (0, 0)),
        ],
        out_specs=pl.BlockSpec((BN, DIM), lambda i: (i, 0)),
        out_shape=jax.ShapeDtypeStruct((n, DIM), jnp.float32),
    )(x, b3, table)

    return out


# speedup vs baseline: 1.3768x; 1.0871x over previous
import jax
import jax.numpy as jnp
import numpy as np
from jax.experimental import pallas as pl
from jax.experimental.pallas import tpu as pltpu

IRREPS = [(128, 0, 1), (64, 1, -1), (32, 2, 1)]
G = 256
EPS = 1e-05
DIM = 480
NMUL = 224
NSC = 128
SW = 640
TW = 640
BN = 1000
W = 40
GP = G + W


def _stats_kernel(x_ref, b_ref, b0_ref, sp_ref, o_ref, acc):
    i = pl.program_id(0)

    @pl.when(i == 0)
    def _():
        acc[...] = jnp.zeros_like(acc)

    x = x_ref[...]
    seg = b_ref[0, 0, :]
    b0a = pl.multiple_of(b0_ref[i], 8)
    span = sp_ref[i]
    ones = jnp.ones((BN, 32), jnp.float32)
    y = jnp.concatenate([x[:, :NSC], x * x, ones], axis=1)

    @pl.when(span <= W - 8)
    def _():
        iota = jax.lax.broadcasted_iota(jnp.int32, (W, BN), 0) + b0a
        oh_t = (iota == seg[None, :]).astype(jnp.float32)
        acc[pl.ds(b0a, W), :] += jax.lax.dot_general(
            oh_t, y, (((1,), (0,)), ((), ())),
            preferred_element_type=jnp.float32)

    @pl.when(span > W - 8)
    def _():
        iota = jax.lax.broadcasted_iota(jnp.int32, (G, BN), 0)
        oh_t = (iota == seg[None, :]).astype(jnp.float32)
        acc[pl.ds(0, G), :] += jax.lax.dot_general(
            oh_t, y, (((1,), (0,)), ((), ())),
            preferred_element_type=jnp.float32)

    @pl.when(i == pl.num_programs(0) - 1)
    def _():
        o_ref[...] = acc[...]


def _table_kernel(st_ref, gm_ref, em_ref, dinv_ref, w_ref, ms_ref, bias_ref,
                  t_ref):
    st = st_ref[:G, :]
    cnt = jnp.maximum(st[:, SW - 32:SW - 31], 1.0)
    s1 = st[:, :NSC]
    sq = st[:, NSC:NSC + DIM]
    m = s1 / cnt
    gq = jax.lax.dot_general(sq, gm_ref[...], (((1,), (0,)), ((), ())),
                             preferred_element_type=jnp.float32)
    ex2 = gq * dinv_ref[...] / cnt
    ms = ms_ref[...]
    corr = (m * m) * (ms * (2.0 - ms))
    corr_p = jnp.concatenate([corr, jnp.zeros((G, NMUL - NSC), jnp.float32)],
                             axis=1)
    fn = jax.lax.rsqrt(ex2 - corr_p + EPS) * w_ref[...]
    scale = jax.lax.dot_general(fn, em_ref[...], (((1,), (0,)), ((), ())),
                                preferred_element_type=jnp.float32)
    off_sc = bias_ref[...] - m * ms * fn[:, :NSC]
    tbl = jnp.concatenate([scale, off_sc,
                           jnp.zeros((G, TW - DIM - NSC), jnp.float32)],
                          axis=1)
    t_ref[...] = jnp.concatenate(
        [tbl, jnp.zeros((GP - G, TW), jnp.float32)], axis=0)


def _apply_kernel(x_ref, b_ref, b0_ref, sp_ref, t_ref, o_ref):
    x = x_ref[...]
    seg = b_ref[0, 0, :]
    b0a = pl.multiple_of(b0_ref[pl.program_id(0)], 8)
    span = sp_ref[pl.program_id(0)]

    def fin(so):
        o_ref[...] = jnp.concatenate(
            [x[:, :NSC] * so[:, :NSC] + so[:, DIM:DIM + NSC],
             x[:, NSC:] * so[:, NSC:DIM]], axis=1)

    @pl.when(span <= W - 8)
    def _():
        iota = jax.lax.broadcasted_iota(jnp.int32, (BN, W), 1) + b0a
        oh = (iota == seg[:, None]).astype(jnp.float32)
        fin(jax.lax.dot_general(oh, t_ref[pl.ds(b0a, W), :],
                                (((1,), (0,)), ((), ())),
                                preferred_element_type=jnp.float32))

    @pl.when(span > W - 8)
    def _():
        iota = jax.lax.broadcasted_iota(jnp.int32, (BN, G), 1)
        oh = (iota == seg[:, None]).astype(jnp.float32)
        fin(jax.lax.dot_general(oh, t_ref[pl.ds(0, G), :],
                                (((1,), (0,)), ((), ())),
                                preferred_element_type=jnp.float32))


def _build_consts():
    d_of = np.concatenate([np.full(mul, 2 * l + 1, np.float32)
                           for mul, l, p in IRREPS])
    gm = np.zeros((DIM, NMUL), np.float32)
    em = np.zeros((NMUL, DIM), np.float32)
    c = 0
    mi = 0
    for mul, l, p in IRREPS:
        d = 2 * l + 1
        for k in range(mul):
            gm[c:c + d, mi] = 1.0
            em[mi, c:c + d] = 1.0
            c += d
            mi += 1
    dinv = (1.0 / d_of)[None, :]
    return gm, em, dinv


_GM, _EM, _DINV = _build_consts()


@jax.jit
def kernel(node_input, batch, mean_shift, affine_weight, affine_bias):
    n = node_input.shape[0]
    nb = n // BN
    x = node_input
    b = batch.astype(jnp.int32)
    b3 = b.reshape(nb, 1, BN)
    first = b3[:, 0, 0]
    last = b3[:, 0, BN - 1]
    b0a = (first // 8) * 8
    span = last - first + 1

    stats = pl.pallas_call(
        _stats_kernel,
        grid=(nb,),
        in_specs=[
            pl.BlockSpec((BN, DIM), lambda i: (i, 0)),
            pl.BlockSpec((1, 1, BN), lambda i: (i, 0, 0)),
            pl.BlockSpec(memory_space=pltpu.SMEM),
            pl.BlockSpec(memory_space=pltpu.SMEM),
        ],
        out_specs=pl.BlockSpec((GP, SW), lambda i: (0, 0)),
        out_shape=jax.ShapeDtypeStruct((GP, SW), jnp.float32),
        scratch_shapes=[pltpu.VMEM((GP, SW), jnp.float32)],
    )(x, b3, b0a, span)

    table = pl.pallas_call(
        _table_kernel,
        out_shape=jax.ShapeDtypeStruct((GP, TW), jnp.float32),
    )(stats, _GM, _EM, _DINV,
      affine_weight[None, :], mean_shift[None, :], affine_bias[None, :])

    out = pl.pallas_call(
        _apply_kernel,
        grid=(nb,),
        in_specs=[
            pl.BlockSpec((BN, DIM), lambda i: (i, 0)),
            pl.BlockSpec((1, 1, BN), lambda i: (i, 0, 0)),
            pl.BlockSpec(memory_space=pltpu.SMEM),
            pl.BlockSpec(memory_space=pltpu.SMEM),
            pl.BlockSpec((GP, TW), lambda i: (0, 0)),
        ],
        out_specs=pl.BlockSpec((BN, DIM), lambda i: (i, 0)),
        out_shape=jax.ShapeDtypeStruct((n, DIM), jnp.float32),
    )(x, b3, b0a, span, table)

    return out
